# baseline (device time: 273743 ns/iter reference)
import numpy as np
import jax
import jax.numpy as jnp
from jax import lax
from jax.experimental import pallas as pl
from jax.experimental.pallas import tpu as pltpu

N_DEV = 4
SQ = 2048
SKV = 2048
DH = 128
HG = 8
DM = 1024
QC = 256
N_QC = SQ // QC
SCALE = 0.08838834764831843
BLK = 64
NBLK = SQ // BLK

_ORDER = [b for rr in range(3) for b in range(NBLK) if b % 3 == rr]
_CLS_SIZES = [len([b for b in range(NBLK) if b % 3 == rr]) for rr in range(3)]
SEG_LEN = tuple(n * BLK for n in _CLS_SIZES)
SEG_START = (0, SEG_LEN[0], SEG_LEN[0] + SEG_LEN[1])
SEG_C = (0, 2, 1)


def _group_blocks(a, axis):
    shape = a.shape
    blocked = a.reshape(shape[:axis] + (NBLK, BLK) + shape[axis + 1:])
    idx = [slice(None)] * axis
    parts = [blocked[tuple(idx + [slice(rr, None, 3)])] for rr in range(3)]
    return jnp.concatenate(parts, axis=axis).reshape(shape)


def kernel(x, Wq, K_ext, V_ext, Wo):
    my = lax.axis_index("i")
    xb = _group_blocks(x[0].astype(jnp.bfloat16), 0)
    wq = Wq.astype(jnp.bfloat16)
    wo = Wo.astype(jnp.bfloat16)

    def body(x_ref, wq_ref, wo_ref, kt_ref, vt_ref, out_ref,
             comm_wq, comm_wo, q_scr, ctx_scr, kg, vg,
             send_sems, recv_sems, credit_sem, kv_sems):
        my_i = lax.axis_index("i")
        left = lax.rem(my_i + N_DEV - 1, N_DEV)
        right = lax.rem(my_i + 1, N_DEV)

        barrier = pltpu.get_barrier_semaphore()
        for nbr in (left, right):
            pl.semaphore_signal(barrier, inc=1, device_id=(nbr,),
                                device_id_type=pl.DeviceIdType.MESH)
        pl.semaphore_wait(barrier, 2)

        rdmas = []
        for j in range(N_DEV):
            o = lax.rem(my_i - j + N_DEV, N_DEV)

            kv_cps = []
            for p, b in enumerate(_ORDER):
                kcp = pltpu.make_async_copy(
                    kt_ref.at[my_i, pl.ds(b * BLK, BLK), pl.ds(o * HG, HG)],
                    kg.at[pl.ds(p * BLK, BLK)], kv_sems.at[0])
                vcp = pltpu.make_async_copy(
                    vt_ref.at[my_i, pl.ds(b * BLK, BLK), pl.ds(o * HG, HG)],
                    vg.at[pl.ds(p * BLK, BLK)], kv_sems.at[1])
                kcp.start()
                vcp.start()
                kv_cps += [kcp, vcp]

            if j < N_DEV - 1:
                if j == 2:
                    pl.semaphore_wait(credit_sem, 1)
                src_wq = wq_ref if j == 0 else comm_wq.at[(j - 1) % 2]
                src_wo = wo_ref if j == 0 else comm_wo.at[(j - 1) % 2]
                r_wq = pltpu.make_async_remote_copy(
                    src_ref=src_wq, dst_ref=comm_wq.at[j % 2],
                    send_sem=send_sems.at[j, 0], recv_sem=recv_sems.at[j, 0],
                    device_id=(right,), device_id_type=pl.DeviceIdType.MESH)
                r_wo = pltpu.make_async_remote_copy(
                    src_ref=src_wo, dst_ref=comm_wo.at[j % 2],
                    send_sem=send_sems.at[j, 1], recv_sem=recv_sems.at[j, 1],
                    device_id=(right,), device_id_type=pl.DeviceIdType.MESH)
                r_wq.start()
                r_wo.start()
                rdmas.append((r_wq, r_wo))

            def qproj_body(qc, carry):
                wq_j = wq_ref[...] if j == 0 else comm_wq[(j - 1) % 2]
                q_scr[pl.ds(qc * QC, QC), :] = (
                    jnp.dot(x_ref[pl.ds(qc * QC, QC), :], wq_j,
                            preferred_element_type=jnp.float32)
                    * SCALE).astype(jnp.bfloat16)
                return carry
            lax.fori_loop(0, N_QC, qproj_body, 0)
            for cp in kv_cps:
                cp.wait()

            for r in range(3):
                sr, lr = SEG_START[r], SEG_LEN[r]
                c = SEG_C[r]
                sc, lc = SEG_START[c], SEG_LEN[c]
                nb = lr // BLK

                def h_body(h, carry, r=r, sr=sr, lr=lr, sc=sc, lc=lc, nb=nb):
                    q_seg = q_scr[pl.ds(sr, lr), pl.ds(h * DH, DH)]
                    k_c = kg[pl.ds(sc, lc), h, :].astype(jnp.bfloat16)
                    v_c = vg[pl.ds(sc, lc), h, :].astype(jnp.bfloat16)
                    e_main = jnp.exp(lax.dot_general(
                        q_seg, k_c, (((1,), (1,)), ((), ())),
                        preferred_element_type=jnp.float32))
                    sm = jnp.sum(e_main, axis=1, keepdims=True)
                    ctx = lax.dot_general(
                        e_main.astype(jnp.bfloat16), v_c,
                        (((1,), (0,)), ((), ())),
                        preferred_element_type=jnp.float32)
                    if r:
                        k0 = kg[pl.ds(0, BLK), h, :].astype(jnp.bfloat16)
                        v0 = vg[pl.ds(0, BLK), h, :].astype(jnp.bfloat16)
                        e0 = jnp.exp(lax.dot_general(
                            q_seg, k0, (((1,), (1,)), ((), ())),
                            preferred_element_type=jnp.float32))
                        sm = sm + jnp.sum(e0, axis=1, keepdims=True)
                        ctx = ctx + lax.dot_general(
                            e0.astype(jnp.bfloat16), v0,
                            (((1,), (0,)), ((), ())),
                            preferred_element_type=jnp.float32)
                        pos = 0
                        sm_parts = []
                        ctx_parts = []
                        while pos < nb:
                            w = min(4, nb - pos)
                            ln = w * BLK
                            qd = q_seg[pos * BLK:pos * BLK + ln, :]
                            kd = kg[pl.ds(sr + pos * BLK, ln), h, :].astype(
                                jnp.bfloat16)
                            vd = vg[pl.ds(sr + pos * BLK, ln), h, :].astype(
                                jnp.bfloat16)
                            ii = lax.broadcasted_iota(jnp.int32, (ln, ln), 0)
                            jj = lax.broadcasted_iota(jnp.int32, (ln, ln), 1)
                            dmask = (ii // BLK == jj // BLK).astype(jnp.float32)
                            ed = jnp.exp(lax.dot_general(
                                qd, kd, (((1,), (1,)), ((), ())),
                                preferred_element_type=jnp.float32)) * dmask
                            sm_parts.append(jnp.sum(ed, axis=1, keepdims=True))
                            ctx_parts.append(lax.dot_general(
                                ed.astype(jnp.bfloat16), vd,
                                (((1,), (0,)), ((), ())),
                                preferred_element_type=jnp.float32))
                            pos += w
                        sm = sm + jnp.concatenate(sm_parts, axis=0)
                        ctx = ctx + jnp.concatenate(ctx_parts, axis=0)
                    ctx_scr[pl.ds(sr, lr), pl.ds(h * DH, DH)] = (
                        ctx * (1.0 / sm)).astype(jnp.bfloat16)
                    return carry
                lax.fori_loop(0, HG, h_body, 0)

            def oproj_body(qc, carry):
                wo_j = wo_ref[...] if j == 0 else comm_wo[(j - 1) % 2]
                oval = jnp.dot(ctx_scr[pl.ds(qc * QC, QC), :], wo_j,
                               preferred_element_type=jnp.float32)
                if j == 0:
                    out_ref[0, pl.ds(qc * QC, QC), :] = oval
                else:
                    out_ref[0, pl.ds(qc * QC, QC), :] = (
                        out_ref[0, pl.ds(qc * QC, QC), :] + oval)
                return carry
            lax.fori_loop(0, N_QC, oproj_body, 0)

            if j < N_DEV - 1:
                r_wq, r_wo = rdmas[j]
                r_wq.wait_send()
                r_wo.wait_send()
                if j == 1:
                    pl.semaphore_signal(credit_sem, inc=1, device_id=(left,),
                                        device_id_type=pl.DeviceIdType.MESH)
                r_wq.wait_recv()
                r_wo.wait_recv()

    out_p = pl.pallas_call(
        body,
        out_shape=jax.ShapeDtypeStruct((1, SQ, DM), jnp.float32),
        in_specs=[
            pl.BlockSpec(memory_space=pltpu.VMEM),
            pl.BlockSpec(memory_space=pltpu.VMEM),
            pl.BlockSpec(memory_space=pltpu.VMEM),
            pl.BlockSpec(memory_space=pl.ANY),
            pl.BlockSpec(memory_space=pl.ANY),
        ],
        out_specs=pl.BlockSpec(memory_space=pltpu.VMEM),
        scratch_shapes=[
            pltpu.VMEM((2, DM, DM), jnp.bfloat16),
            pltpu.VMEM((2, DM, DM), jnp.bfloat16),
            pltpu.VMEM((SQ, DM), jnp.bfloat16),
            pltpu.VMEM((SQ, DM), jnp.bfloat16),
            pltpu.VMEM((SKV, HG, DH), jnp.float32),
            pltpu.VMEM((SKV, HG, DH), jnp.float32),
            pltpu.SemaphoreType.DMA((N_DEV - 1, 2)),
            pltpu.SemaphoreType.DMA((N_DEV - 1, 2)),
            pltpu.SemaphoreType.REGULAR,
            pltpu.SemaphoreType.DMA((2,)),
        ],
        compiler_params=pltpu.CompilerParams(
            collective_id=0, vmem_limit_bytes=36 * 1024 * 1024),
    )(xb, wq, wo, K_ext, V_ext)
    op = out_p[0].reshape(NBLK, BLK, DM)
    n0, n1, n2 = _CLS_SIZES
    s0, s1, s2 = op[:n0], op[n0:n0 + n1], op[n0 + n1:]
    main = jnp.stack([s0[:n2], s1[:n2], s2], axis=1).reshape(3 * n2 * BLK, DM)
    tail = jnp.concatenate([s0[n2:], s1[n2:]], axis=0).reshape(-1, DM)
    return jnp.concatenate([main, tail], axis=0)[None]


# device time: 246110 ns/iter; 1.1123x vs baseline; 1.1123x over previous
import numpy as np
import jax
import jax.numpy as jnp
from jax import lax
from jax.experimental import pallas as pl
from jax.experimental.pallas import tpu as pltpu

N_DEV = 4
SQ = 2048
SKV = 2048
DH = 128
HG = 8
DM = 1024
QC = 256
N_QC = SQ // QC
SCALE = 0.08838834764831843
BLK = 64
NBLK = SQ // BLK

_ORDER = [b for rr in range(3) for b in range(NBLK) if b % 3 == rr]
_CLS_SIZES = [len([b for b in range(NBLK) if b % 3 == rr]) for rr in range(3)]
SEG_LEN = tuple(n * BLK for n in _CLS_SIZES)
SEG_START = (0, SEG_LEN[0], SEG_LEN[0] + SEG_LEN[1])
SEG_C = (0, 2, 1)


def _group_blocks(a, axis):
    shape = a.shape
    blocked = a.reshape(shape[:axis] + (NBLK, BLK) + shape[axis + 1:])
    idx = [slice(None)] * axis
    parts = [blocked[tuple(idx + [slice(rr, None, 3)])] for rr in range(3)]
    return jnp.concatenate(parts, axis=axis).reshape(shape)


def kernel(x, Wq, K_ext, V_ext, Wo):
    my = lax.axis_index("i")
    xb = _group_blocks(x[0].astype(jnp.bfloat16), 0)
    wq = Wq.astype(jnp.bfloat16)
    wo = Wo.astype(jnp.bfloat16)

    def body(x_ref, wq_ref, wo_ref, kt_ref, vt_ref, out_ref,
             comm_wq, comm_wo, q_scr, ctx_scr, kg, vg,
             send_sems, recv_sems, credit_sem, kv_sems):
        my_i = lax.axis_index("i")
        left = lax.rem(my_i + N_DEV - 1, N_DEV)
        right = lax.rem(my_i + 1, N_DEV)

        barrier = pltpu.get_barrier_semaphore()
        for nbr in (left, right):
            pl.semaphore_signal(barrier, inc=1, device_id=(nbr,),
                                device_id_type=pl.DeviceIdType.MESH)
        pl.semaphore_wait(barrier, 2)

        rdmas = []
        for j in range(N_DEV):
            o = lax.rem(my_i - j + N_DEV, N_DEV)

            kv_cps = []
            for p, b in enumerate(_ORDER):
                kcp = pltpu.make_async_copy(
                    kt_ref.at[my_i, pl.ds(b * BLK, BLK), pl.ds(o * HG, HG)],
                    kg.at[pl.ds(p * BLK, BLK)], kv_sems.at[0])
                vcp = pltpu.make_async_copy(
                    vt_ref.at[my_i, pl.ds(b * BLK, BLK), pl.ds(o * HG, HG)],
                    vg.at[pl.ds(p * BLK, BLK)], kv_sems.at[1])
                kcp.start()
                vcp.start()
                kv_cps += [kcp, vcp]

            if j < N_DEV - 1:
                if j == 2:
                    pl.semaphore_wait(credit_sem, 1)
                src_wq = wq_ref if j == 0 else comm_wq.at[(j - 1) % 2]
                src_wo = wo_ref if j == 0 else comm_wo.at[(j - 1) % 2]
                r_wq = pltpu.make_async_remote_copy(
                    src_ref=src_wq, dst_ref=comm_wq.at[j % 2],
                    send_sem=send_sems.at[j, 0], recv_sem=recv_sems.at[j, 0],
                    device_id=(right,), device_id_type=pl.DeviceIdType.MESH)
                r_wo = pltpu.make_async_remote_copy(
                    src_ref=src_wo, dst_ref=comm_wo.at[j % 2],
                    send_sem=send_sems.at[j, 1], recv_sem=recv_sems.at[j, 1],
                    device_id=(right,), device_id_type=pl.DeviceIdType.MESH)
                r_wq.start()
                r_wo.start()
                rdmas.append((r_wq, r_wo))

            def qproj_body(qc, carry):
                wq_j = wq_ref[...] if j == 0 else comm_wq[(j - 1) % 2]
                q_scr[pl.ds(qc * QC, QC), :] = (
                    jnp.dot(x_ref[pl.ds(qc * QC, QC), :], wq_j,
                            preferred_element_type=jnp.float32)
                    * SCALE).astype(jnp.bfloat16)
                return carry
            lax.fori_loop(0, N_QC, qproj_body, 0)
            for cp in kv_cps:
                cp.wait()

            for r in range(3):
                sr, lr = SEG_START[r], SEG_LEN[r]
                c = SEG_C[r]
                sc, lc = SEG_START[c], SEG_LEN[c]
                nb = lr // BLK

                def h_body(h, carry, r=r, sr=sr, lr=lr, sc=sc, lc=lc, nb=nb):
                    q_seg = q_scr[pl.ds(sr, lr), pl.ds(h * DH, DH)]
                    k_c = kg[pl.ds(sc, lc), h, :].astype(jnp.bfloat16)
                    v_c = vg[pl.ds(sc, lc), h, :].astype(jnp.bfloat16)
                    e_main = jnp.exp(lax.dot_general(
                        q_seg, k_c, (((1,), (1,)), ((), ())),
                        preferred_element_type=jnp.float32))
                    sm = jnp.sum(e_main, axis=1, keepdims=True)
                    ctx = lax.dot_general(
                        e_main.astype(jnp.bfloat16), v_c,
                        (((1,), (0,)), ((), ())),
                        preferred_element_type=jnp.float32)
                    if r:
                        k0 = kg[pl.ds(0, BLK), h, :].astype(jnp.bfloat16)
                        v0 = vg[pl.ds(0, BLK), h, :].astype(jnp.bfloat16)
                        e0 = jnp.exp(lax.dot_general(
                            q_seg, k0, (((1,), (1,)), ((), ())),
                            preferred_element_type=jnp.float32))
                        sm = sm + jnp.sum(e0, axis=1, keepdims=True)
                        ctx = ctx + lax.dot_general(
                            e0.astype(jnp.bfloat16), v0,
                            (((1,), (0,)), ((), ())),
                            preferred_element_type=jnp.float32)
                        q3 = q_seg.reshape(nb, BLK, DH)
                        k3 = kg[pl.ds(sr, lr), h, :].astype(
                            jnp.bfloat16).reshape(nb, BLK, DH)
                        v3 = vg[pl.ds(sr, lr), h, :].astype(
                            jnp.bfloat16).reshape(nb, BLK, DH)
                        ed = jnp.exp(lax.dot_general(
                            q3, k3, (((2,), (2,)), ((0,), (0,))),
                            preferred_element_type=jnp.float32))
                        sm = sm + jnp.sum(ed, axis=2).reshape(lr, 1)
                        ctx = ctx + lax.dot_general(
                            ed.astype(jnp.bfloat16), v3,
                            (((2,), (1,)), ((0,), (0,))),
                            preferred_element_type=jnp.float32).reshape(lr, DH)
                    ctx_scr[pl.ds(sr, lr), pl.ds(h * DH, DH)] = (
                        ctx * (1.0 / sm)).astype(jnp.bfloat16)
                    return carry
                lax.fori_loop(0, HG, h_body, 0)

            def oproj_body(qc, carry):
                wo_j = wo_ref[...] if j == 0 else comm_wo[(j - 1) % 2]
                oval = jnp.dot(ctx_scr[pl.ds(qc * QC, QC), :], wo_j,
                               preferred_element_type=jnp.float32)
                if j == 0:
                    out_ref[0, pl.ds(qc * QC, QC), :] = oval
                else:
                    out_ref[0, pl.ds(qc * QC, QC), :] = (
                        out_ref[0, pl.ds(qc * QC, QC), :] + oval)
                return carry
            lax.fori_loop(0, N_QC, oproj_body, 0)

            if j < N_DEV - 1:
                r_wq, r_wo = rdmas[j]
                r_wq.wait_send()
                r_wo.wait_send()
                if j == 1:
                    pl.semaphore_signal(credit_sem, inc=1, device_id=(left,),
                                        device_id_type=pl.DeviceIdType.MESH)
                r_wq.wait_recv()
                r_wo.wait_recv()

    out_p = pl.pallas_call(
        body,
        out_shape=jax.ShapeDtypeStruct((1, SQ, DM), jnp.float32),
        in_specs=[
            pl.BlockSpec(memory_space=pltpu.VMEM),
            pl.BlockSpec(memory_space=pltpu.VMEM),
            pl.BlockSpec(memory_space=pltpu.VMEM),
            pl.BlockSpec(memory_space=pl.ANY),
            pl.BlockSpec(memory_space=pl.ANY),
        ],
        out_specs=pl.BlockSpec(memory_space=pltpu.VMEM),
        scratch_shapes=[
            pltpu.VMEM((2, DM, DM), jnp.bfloat16),
            pltpu.VMEM((2, DM, DM), jnp.bfloat16),
            pltpu.VMEM((SQ, DM), jnp.bfloat16),
            pltpu.VMEM((SQ, DM), jnp.bfloat16),
            pltpu.VMEM((SKV, HG, DH), jnp.float32),
            pltpu.VMEM((SKV, HG, DH), jnp.float32),
            pltpu.SemaphoreType.DMA((N_DEV - 1, 2)),
            pltpu.SemaphoreType.DMA((N_DEV - 1, 2)),
            pltpu.SemaphoreType.REGULAR,
            pltpu.SemaphoreType.DMA((2,)),
        ],
        compiler_params=pltpu.CompilerParams(
            collective_id=0, vmem_limit_bytes=36 * 1024 * 1024),
    )(xb, wq, wo, K_ext, V_ext)
    op = out_p[0].reshape(NBLK, BLK, DM)
    n0, n1, n2 = _CLS_SIZES
    s0, s1, s2 = op[:n0], op[n0:n0 + n1], op[n0 + n1:]
    main = jnp.stack([s0[:n2], s1[:n2], s2], axis=1).reshape(3 * n2 * BLK, DM)
    tail = jnp.concatenate([s0[n2:], s1[n2:]], axis=0).reshape(-1, DM)
    return jnp.concatenate([main, tail], axis=0)[None]
